# baseline (speedup 1.0000x reference)
import functools

import jax
import jax.numpy as jnp
from jax import lax
from jax.experimental import pallas as pl
from jax.experimental.pallas import tpu as pltpu
from jax.experimental.pallas import tpu_sc as plsc

CHUNK = 128
NBUF = 4
BB = 64
EPS = 1e-12

_info = plsc.get_sparse_core_info()
_NC, _NS = _info.num_cores, _info.num_subcores
NW = _NC * _NS


def _make_sc_gather(N, H):
    per_w = N // NW
    nchunks = per_w // CHUNK
    assert per_w % CHUNK == 0 and nchunks >= NBUF + 2

    def body(ids_hbm, word_hbm, out_hbm, ids_v, rows_v, *sems):
        sem_g = sems[:NBUF]
        sem_o = sems[NBUF:]
        wid = lax.axis_index("s") * _NC + lax.axis_index("c")
        w0 = wid * per_w
        pltpu.sync_copy(ids_hbm.at[wid], ids_v)

        def gather(c, b):
            pltpu.async_copy(word_hbm.at[ids_v.at[c]], rows_v.at[b], sem_g[b])

        def gather_wait(b):
            pltpu.make_async_copy(word_hbm.at[ids_v.at[0]], rows_v.at[b],
                                  sem_g[b]).wait()

        def put(c, b):
            pltpu.async_copy(rows_v.at[b],
                             out_hbm.at[pl.ds(w0 + c * CHUNK, CHUNK)],
                             sem_o[b])

        def put_wait(b):
            pltpu.make_async_copy(rows_v.at[b], out_hbm.at[pl.ds(0, CHUNK)],
                                  sem_o[b]).wait()

        gather(0, 0)
        gather(1, 1)

        nmain = (nchunks - 2) // NBUF * NBUF

        def quad_body(i, carry):
            for k in range(NBUF):
                c = i * NBUF + k
                gather_wait(k)
                put(c, k)
                b2 = (k + 2) % NBUF
                if k >= 2:
                    put_wait(b2)
                else:
                    @pl.when(i >= 1)
                    def _():
                        put_wait(b2)
                gather(c + 2, b2)
            return carry

        lax.fori_loop(0, nmain // NBUF, quad_body, 0)
        for c in range(nmain, nchunks):
            b = c % NBUF
            gather_wait(b)
            put(c, b)
            if c + 2 < nchunks:
                b2 = (c + 2) % NBUF
                put_wait(b2)
                gather(c + 2, b2)
        for c in range(nchunks - NBUF, nchunks):
            put_wait(c % NBUF)

    return pl.kernel(
        body,
        out_type=jax.ShapeDtypeStruct((N, H), jnp.float32),
        mesh=plsc.VectorSubcoreMesh(core_axis_name="c", subcore_axis_name="s"),
        scratch_types=[
            pltpu.VMEM((nchunks, CHUNK), jnp.int32),
            pltpu.VMEM((NBUF, CHUNK, H), jnp.float32),
        ] + [pltpu.SemaphoreType.DMA] * (2 * NBUF),
    )


def _tc_body(x_ref, tt_ref, pos_ref, ty_ref, gb_ref, o_ref):
    x = x_ref[...]
    ttf = tt_ref[...].astype(jnp.float32)
    pos = pos_ref[...]
    tdiff = ty_ref[1] - ty_ref[0]
    gamma = gb_ref[0]
    beta = gb_ref[1]
    x = x + (pos[None, :, :] + ty_ref[0]) + ttf * tdiff
    mean = jnp.mean(x, axis=-1, keepdims=True)
    xc = x - mean
    var = jnp.mean(xc * xc, axis=-1, keepdims=True)
    o_ref[...] = xc * lax.rsqrt(var + EPS) * gamma + beta


def _tc_ln(B, S, H):
    grid = (B // BB,)
    return pl.pallas_call(
        _tc_body,
        grid=grid,
        in_specs=[
            pl.BlockSpec((BB, S, H), lambda i: (i, 0, 0)),
            pl.BlockSpec((BB, S, 1), lambda i: (i, 0, 0)),
            pl.BlockSpec((S, H), lambda i: (0, 0)),
            pl.BlockSpec((2, H), lambda i: (0, 0)),
            pl.BlockSpec((2, H), lambda i: (0, 0)),
        ],
        out_specs=pl.BlockSpec((BB, S, H), lambda i: (i, 0, 0)),
        out_shape=jax.ShapeDtypeStruct((B, S, H), jnp.float32),
    )


SPLITS = 2


def kernel(input_ids, token_type_ids, word_table, pos_table, type_table,
           ln_gamma, ln_beta):
    B, S = input_ids.shape
    H = word_table.shape[1]
    N = B * S
    Bh = B // SPLITS
    Nh = N // SPLITS
    ids = input_ids.reshape(SPLITS, NW, Nh // (NW * CHUNK),
                            CHUNK).astype(jnp.int32)
    tt = token_type_ids.astype(jnp.int32)[:, :, None]
    gb = jnp.stack([ln_gamma, ln_beta])
    pos = pos_table[:S]
    sc = _make_sc_gather(Nh, H)
    ln = _tc_ln(Bh, S, H)
    outs = []
    for h in range(SPLITS):
        g = sc(ids[h], word_table).reshape(Bh, S, H)
        outs.append(ln(g, tt[h * Bh:(h + 1) * Bh], pos, type_table, gb))
    return jnp.concatenate(outs, axis=0)


# speedup vs baseline: 1.2313x; 1.2313x over previous
import functools

import jax
import jax.numpy as jnp
from jax import lax
from jax.experimental import pallas as pl
from jax.experimental.pallas import tpu as pltpu
from jax.experimental.pallas import tpu_sc as plsc

CHUNK = 128
NBUF = 6
DEPTH = 3
BB = 32
EPS = 1e-12

_info = plsc.get_sparse_core_info()
_NC, _NS = _info.num_cores, _info.num_subcores
NW = _NC * _NS


def _make_sc_gather(N, H):
    per_w = N // NW
    nchunks = per_w // CHUNK
    assert per_w % CHUNK == 0 and nchunks >= NBUF + DEPTH

    def body(ids_hbm, word_hbm, out_hbm, ids_v, rows_v, *sems):
        sem_g = sems[:NBUF]
        sem_o = sems[NBUF:]
        wid = lax.axis_index("s") * _NC + lax.axis_index("c")
        w0 = wid * per_w
        pltpu.sync_copy(ids_hbm.at[wid], ids_v)

        def gather(c, b):
            pltpu.async_copy(word_hbm.at[ids_v.at[c]], rows_v.at[b], sem_g[b])

        def gather_wait(b):
            pltpu.make_async_copy(word_hbm.at[ids_v.at[0]], rows_v.at[b],
                                  sem_g[b]).wait()

        def put(c, b):
            pltpu.async_copy(rows_v.at[b],
                             out_hbm.at[pl.ds(w0 + c * CHUNK, CHUNK)],
                             sem_o[b])

        def put_wait(b):
            pltpu.make_async_copy(rows_v.at[b], out_hbm.at[pl.ds(0, CHUNK)],
                                  sem_o[b]).wait()

        for c in range(DEPTH):
            gather(c, c)

        nmain = (nchunks - DEPTH) // NBUF * NBUF

        def ring_body(i, carry):
            for k in range(NBUF):
                c = i * NBUF + k
                gather_wait(k)
                put(c, k)
                b2 = (k + DEPTH) % NBUF
                if k >= DEPTH:
                    put_wait(b2)
                else:
                    @pl.when(i >= 1)
                    def _():
                        put_wait(b2)
                gather(c + DEPTH, b2)
            return carry

        lax.fori_loop(0, nmain // NBUF, ring_body, 0)
        for c in range(nmain, nchunks):
            b = c % NBUF
            gather_wait(b)
            put(c, b)
            if c + DEPTH < nchunks:
                b2 = (c + DEPTH) % NBUF
                put_wait(b2)
                gather(c + DEPTH, b2)
        for c in range(nchunks - NBUF, nchunks):
            put_wait(c % NBUF)

    return pl.kernel(
        body,
        out_type=jax.ShapeDtypeStruct((N, H), jnp.float32),
        mesh=plsc.VectorSubcoreMesh(core_axis_name="c", subcore_axis_name="s"),
        scratch_types=[
            pltpu.VMEM((nchunks, CHUNK), jnp.int32),
            pltpu.VMEM((NBUF, CHUNK, H), jnp.float32),
        ] + [pltpu.SemaphoreType.DMA] * (2 * NBUF),
    )


def _tc_body(x_ref, tt_ref, pos_ref, ty_ref, gb_ref, o_ref):
    x = x_ref[...]
    ttf = tt_ref[...].astype(jnp.float32)
    pos = pos_ref[...]
    tdiff = ty_ref[1] - ty_ref[0]
    gamma = gb_ref[0]
    beta = gb_ref[1]
    x = x + (pos[None, :, :] + ty_ref[0]) + ttf * tdiff
    inv_h = 1.0 / x.shape[-1]
    m = jnp.sum(x, axis=-1, keepdims=True) * inv_h
    q = jnp.sum(x * x, axis=-1, keepdims=True) * inv_h
    var = jnp.maximum(q - m * m, 0.0)
    rg = lax.rsqrt(var + EPS) * gamma
    o_ref[...] = (x - m) * rg + beta


def _tc_ln(B, S, H):
    grid = (B // BB,)
    return pl.pallas_call(
        _tc_body,
        grid=grid,
        in_specs=[
            pl.BlockSpec((BB, S, H), lambda i: (i, 0, 0)),
            pl.BlockSpec((BB, S, 1), lambda i: (i, 0, 0)),
            pl.BlockSpec((S, H), lambda i: (0, 0)),
            pl.BlockSpec((2, H), lambda i: (0, 0)),
            pl.BlockSpec((2, H), lambda i: (0, 0)),
        ],
        out_specs=pl.BlockSpec((BB, S, H), lambda i: (i, 0, 0)),
        out_shape=jax.ShapeDtypeStruct((B, S, H), jnp.float32),
    )


SPLITS = 1


def kernel(input_ids, token_type_ids, word_table, pos_table, type_table,
           ln_gamma, ln_beta):
    B, S = input_ids.shape
    H = word_table.shape[1]
    N = B * S
    Bh = B // SPLITS
    Nh = N // SPLITS
    ids = input_ids.reshape(SPLITS, NW, Nh // (NW * CHUNK),
                            CHUNK).astype(jnp.int32)
    tt = token_type_ids.astype(jnp.int32)[:, :, None]
    gb = jnp.stack([ln_gamma, ln_beta])
    pos = pos_table[:S]
    sc = _make_sc_gather(Nh, H)
    ln = _tc_ln(Bh, S, H)
    outs = []
    for h in range(SPLITS):
        g = sc(ids[h], word_table).reshape(Bh, S, H)
        outs.append(ln(g, tt[h * Bh:(h + 1) * Bh], pos, type_table, gb))
    return jnp.concatenate(outs, axis=0)


# speedup vs baseline: 1.3036x; 1.0587x over previous
import functools

import jax
import jax.numpy as jnp
from jax import lax
from jax.experimental import pallas as pl
from jax.experimental.pallas import tpu as pltpu
from jax.experimental.pallas import tpu_sc as plsc

CHUNK = 128
NBUF = 6
DEPTH = 3
BB = 64
EPS = 1e-12

_info = plsc.get_sparse_core_info()
_NC, _NS = _info.num_cores, _info.num_subcores
NW = _NC * _NS


def _make_sc_gather(N, H):
    per_w = N // NW
    nchunks = per_w // CHUNK
    assert per_w % CHUNK == 0 and nchunks >= NBUF + DEPTH

    def body(ids_hbm, word_hbm, out_hbm, ids_v, rows_v, *sems):
        sem_g = sems[:NBUF]
        sem_o = sems[NBUF:]
        wid = lax.axis_index("s") * _NC + lax.axis_index("c")
        w0 = wid * per_w
        pltpu.sync_copy(ids_hbm.at[wid], ids_v)

        def gather(c, b):
            pltpu.async_copy(word_hbm.at[ids_v.at[c]], rows_v.at[b], sem_g[b])

        def gather_wait(b):
            pltpu.make_async_copy(word_hbm.at[ids_v.at[0]], rows_v.at[b],
                                  sem_g[b]).wait()

        def put(c, b):
            pltpu.async_copy(rows_v.at[b],
                             out_hbm.at[pl.ds(w0 + c * CHUNK, CHUNK)],
                             sem_o[b])

        def put_wait(b):
            pltpu.make_async_copy(rows_v.at[b], out_hbm.at[pl.ds(0, CHUNK)],
                                  sem_o[b]).wait()

        for c in range(DEPTH):
            gather(c, c)

        nmain = (nchunks - DEPTH) // NBUF * NBUF

        def ring_body(i, carry):
            for k in range(NBUF):
                c = i * NBUF + k
                gather_wait(k)
                put(c, k)
                b2 = (k + DEPTH) % NBUF
                if k >= DEPTH:
                    put_wait(b2)
                else:
                    @pl.when(i >= 1)
                    def _():
                        put_wait(b2)
                gather(c + DEPTH, b2)
            return carry

        lax.fori_loop(0, nmain // NBUF, ring_body, 0)
        for c in range(nmain, nchunks):
            b = c % NBUF
            gather_wait(b)
            put(c, b)
            if c + DEPTH < nchunks:
                b2 = (c + DEPTH) % NBUF
                put_wait(b2)
                gather(c + DEPTH, b2)
        for c in range(nchunks - NBUF, nchunks):
            put_wait(c % NBUF)

    return pl.kernel(
        body,
        out_type=jax.ShapeDtypeStruct((N, H), jnp.float32),
        mesh=plsc.VectorSubcoreMesh(core_axis_name="c", subcore_axis_name="s"),
        scratch_types=[
            pltpu.VMEM((nchunks, CHUNK), jnp.int32),
            pltpu.VMEM((NBUF, CHUNK, H), jnp.float32),
        ] + [pltpu.SemaphoreType.DMA] * (2 * NBUF),
    )


def _tc_body(x_ref, tt_ref, pos_ref, ty_ref, gb_ref, o_ref):
    x = x_ref[...]
    ttf = tt_ref[...].astype(jnp.float32)
    pos = pos_ref[...]
    tdiff = ty_ref[1] - ty_ref[0]
    gamma = gb_ref[0]
    beta = gb_ref[1]
    x = x + (pos[None, :, :] + ty_ref[0]) + ttf * tdiff
    mean = jnp.mean(x, axis=-1, keepdims=True)
    xc = x - mean
    var = jnp.mean(xc * xc, axis=-1, keepdims=True)
    o_ref[...] = xc * lax.rsqrt(var + EPS) * gamma + beta


def _tc_ln(B, S, H):
    grid = (B // BB,)
    return pl.pallas_call(
        _tc_body,
        grid=grid,
        in_specs=[
            pl.BlockSpec((BB, S, H), lambda i: (i, 0, 0)),
            pl.BlockSpec((BB, S, 1), lambda i: (i, 0, 0)),
            pl.BlockSpec((S, H), lambda i: (0, 0)),
            pl.BlockSpec((2, H), lambda i: (0, 0)),
            pl.BlockSpec((2, H), lambda i: (0, 0)),
        ],
        out_specs=pl.BlockSpec((BB, S, H), lambda i: (i, 0, 0)),
        out_shape=jax.ShapeDtypeStruct((B, S, H), jnp.float32),
    )


SPLITS = 1


def kernel(input_ids, token_type_ids, word_table, pos_table, type_table,
           ln_gamma, ln_beta):
    B, S = input_ids.shape
    H = word_table.shape[1]
    N = B * S
    Bh = B // SPLITS
    Nh = N // SPLITS
    ids = input_ids.reshape(SPLITS, NW, Nh // (NW * CHUNK),
                            CHUNK).astype(jnp.int32)
    tt = token_type_ids.astype(jnp.int32)[:, :, None]
    gb = jnp.stack([ln_gamma, ln_beta])
    pos = pos_table[:S]
    sc = _make_sc_gather(Nh, H)
    ln = _tc_ln(Bh, S, H)
    outs = []
    for h in range(SPLITS):
        g = sc(ids[h], word_table).reshape(Bh, S, H)
        outs.append(ln(g, tt[h * Bh:(h + 1) * Bh], pos, type_table, gb))
    return jnp.concatenate(outs, axis=0)


# speedup vs baseline: 1.6650x; 1.2772x over previous
import functools

import jax
import jax.numpy as jnp
from jax import lax
from jax.experimental import pallas as pl
from jax.experimental.pallas import tpu as pltpu
from jax.experimental.pallas import tpu_sc as plsc

CHUNK = 128
NBUF = 6
DEPTH = 3
BB = 64
EPS = 1e-12

_info = plsc.get_sparse_core_info()
_NC, _NS = _info.num_cores, _info.num_subcores
NW = _NC * _NS


def _make_sc_gather(N, H):
    per_w = N // NW
    nchunks = per_w // CHUNK
    assert per_w % CHUNK == 0 and nchunks >= NBUF + DEPTH

    def body(ids_hbm, word_hbm, out_hbm, ids_v, rows_v, *sems):
        sem_g = sems[:NBUF]
        sem_o = sems[NBUF:]
        wid = lax.axis_index("s") * _NC + lax.axis_index("c")
        w0 = wid * per_w
        pltpu.sync_copy(ids_hbm.at[wid], ids_v)

        def gather(c, b):
            pltpu.async_copy(word_hbm.at[ids_v.at[c]], rows_v.at[b], sem_g[b])

        def gather_wait(b):
            pltpu.make_async_copy(word_hbm.at[ids_v.at[0]], rows_v.at[b],
                                  sem_g[b]).wait()

        def put(c, b):
            pltpu.async_copy(rows_v.at[b],
                             out_hbm.at[pl.ds(w0 + c * CHUNK, CHUNK)],
                             sem_o[b])

        def put_wait(b):
            pltpu.make_async_copy(rows_v.at[b], out_hbm.at[pl.ds(0, CHUNK)],
                                  sem_o[b]).wait()

        for c in range(DEPTH):
            gather(c, c)

        nmain = (nchunks - DEPTH) // NBUF * NBUF

        def ring_body(i, carry):
            for k in range(NBUF):
                c = i * NBUF + k
                gather_wait(k)
                put(c, k)
                b2 = (k + DEPTH) % NBUF
                if k >= DEPTH:
                    put_wait(b2)
                else:
                    @pl.when(i >= 1)
                    def _():
                        put_wait(b2)
                gather(c + DEPTH, b2)
            return carry

        lax.fori_loop(0, nmain // NBUF, ring_body, 0)
        for c in range(nmain, nchunks):
            b = c % NBUF
            gather_wait(b)
            put(c, b)
            if c + DEPTH < nchunks:
                b2 = (c + DEPTH) % NBUF
                put_wait(b2)
                gather(c + DEPTH, b2)
        for c in range(nchunks - NBUF, nchunks):
            put_wait(c % NBUF)

    return pl.kernel(
        body,
        out_type=jax.ShapeDtypeStruct((N, H), jnp.float32),
        mesh=plsc.VectorSubcoreMesh(core_axis_name="c", subcore_axis_name="s"),
        scratch_types=[
            pltpu.VMEM((nchunks, CHUNK), jnp.int32),
            pltpu.VMEM((NBUF, CHUNK, H), jnp.float32),
        ] + [pltpu.SemaphoreType.DMA] * (2 * NBUF),
    )


def _tc_body(x_ref, tt_ref, pos_ref, ty_ref, gb_ref, o_ref):
    x = x_ref[...]
    tt2 = tt_ref[...]
    ttf = lax.broadcast_in_dim(tt2, x.shape, (0, 1))
    pos = pos_ref[...]
    tdiff = ty_ref[1] - ty_ref[0]
    gamma = gb_ref[0]
    beta = gb_ref[1]
    x = x + (pos[None, :, :] + ty_ref[0]) + ttf * tdiff
    mean = jnp.mean(x, axis=-1, keepdims=True)
    xc = x - mean
    var = jnp.mean(xc * xc, axis=-1, keepdims=True)
    o_ref[...] = xc * lax.rsqrt(var + EPS) * gamma + beta


def _tc_ln(B, S, H):
    grid = (B // BB,)
    return pl.pallas_call(
        _tc_body,
        grid=grid,
        in_specs=[
            pl.BlockSpec((BB, S, H), lambda i: (i, 0, 0)),
            pl.BlockSpec((BB, S), lambda i: (i, 0)),
            pl.BlockSpec((S, H), lambda i: (0, 0)),
            pl.BlockSpec((2, H), lambda i: (0, 0)),
            pl.BlockSpec((2, H), lambda i: (0, 0)),
        ],
        out_specs=pl.BlockSpec((BB, S, H), lambda i: (i, 0, 0)),
        out_shape=jax.ShapeDtypeStruct((B, S, H), jnp.float32),
    )


SPLITS = 1


def kernel(input_ids, token_type_ids, word_table, pos_table, type_table,
           ln_gamma, ln_beta):
    B, S = input_ids.shape
    H = word_table.shape[1]
    N = B * S
    Bh = B // SPLITS
    Nh = N // SPLITS
    ids = input_ids.reshape(SPLITS, NW, Nh // (NW * CHUNK),
                            CHUNK).astype(jnp.int32)
    tt = token_type_ids.astype(jnp.float32)
    gb = jnp.stack([ln_gamma, ln_beta])
    pos = pos_table[:S]
    sc = _make_sc_gather(Nh, H)
    ln = _tc_ln(Bh, S, H)
    outs = []
    for h in range(SPLITS):
        g = sc(ids[h], word_table).reshape(Bh, S, H)
        outs.append(ln(g, tt[h * Bh:(h + 1) * Bh], pos, type_table, gb))
    return jnp.concatenate(outs, axis=0)
